# baseline (speedup 1.0000x reference)
import functools
import jax
import jax.numpy as jnp
from jax import lax
from jax.experimental import pallas as pl
from jax.experimental.pallas import tpu as pltpu
from jax.experimental.pallas import tpu_sc as plsc

C = 128
S = 512
STEPS = 3
T = 512
NW = 32
SPW = S // NW
CH = 256
N_ROWS = 100000
NEG = -1e30


SLICE = 3136
NPAD = 32 * SLICE
NVEC = SLICE // 16


def _sc_off_scan(b_hbm, bnd_hbm, ibuf, bnd):
    wid = lax.axis_index("s") * 2 + lax.axis_index("c")
    lane = lax.broadcasted_iota(jnp.int32, (16,), 0)
    nfull = jnp.full((16,), N_ROWS, jnp.int32)
    for j in range(34):
        bnd[pl.ds(16 * j, 16)] = nfull

    base0 = SLICE * wid - 16

    @pl.when(wid > 0)
    def _ld():
        pltpu.sync_copy(b_hbm.at[pl.ds(base0, 16 + SLICE)], ibuf)

    @pl.when(wid == 0)
    def _ld0():
        ibuf[pl.ds(0, 16)] = jnp.full((16,), -1, jnp.int32)
        pltpu.sync_copy(b_hbm.at[pl.ds(0, SLICE)], ibuf.at[pl.ds(16, SLICE)])

    def vec(j, _):
        v = ibuf[pl.ds(16 + j * 16, 16)]
        sv = ibuf[pl.ds(15 + j * 16, 16)]
        pos = jnp.broadcast_to(base0 + 16 + j * 16, (16,)).astype(jnp.int32) + lane
        plsc.store_scatter(bnd, [v], pos, mask=v != sv)
        return 0

    lax.fori_loop(0, NVEC, vec, 0)
    pltpu.sync_copy(bnd, bnd_hbm.at[wid])


def _sc_off_combine(bnd_hbm, off_hbm, cmb, offv):
    wid = lax.axis_index("s") * 2 + lax.axis_index("c")

    @pl.when(wid == 0)
    def _combine():
        pltpu.sync_copy(bnd_hbm, cmb)
        carry = jnp.int32(N_ROWS)
        for j in reversed(range(34)):
            v = cmb[0, pl.ds(16 * j, 16)]
            for row in range(1, 32):
                v = jnp.minimum(v, cmb[row, pl.ds(16 * j, 16)])
            pm = -plsc.cummax(-lax.rev(v, (0,)))
            pm2 = jnp.minimum(pm, jnp.broadcast_to(carry, (16,)))
            carry = pm2[15]
            offv[pl.ds(16 * j, 16)] = lax.rev(pm2, (0,))
        pltpu.sync_copy(offv, off_hbm)


def _sc_offsets(batch_p):
    mesh = plsc.VectorSubcoreMesh(core_axis_name="c", subcore_axis_name="s")
    scan = functools.partial(
        pl.kernel,
        mesh=mesh,
        compiler_params=pltpu.CompilerParams(needs_layout_passes=False),
        out_type=jax.ShapeDtypeStruct((32, 544), jnp.int32),
        scratch_types=[
            pltpu.VMEM((16 + SLICE,), jnp.int32),
            pltpu.VMEM((544,), jnp.int32),
        ],
    )(_sc_off_scan)
    combine = functools.partial(
        pl.kernel,
        mesh=mesh,
        compiler_params=pltpu.CompilerParams(needs_layout_passes=False),
        out_type=jax.ShapeDtypeStruct((544,), jnp.int32),
        scratch_types=[
            pltpu.VMEM((32, 544), jnp.int32),
            pltpu.VMEM((544,), jnp.int32),
        ],
    )(_sc_off_combine)
    return combine(scan(batch_p))


def _gru_body(qs_ref, h_ref, wih_ref, whh_ref, bih_ref, bhh_ref, out_ref):
    qs, h = qs_ref[...], h_ref[...]
    gi = lax.dot_general(qs, wih_ref[...], (((1,), (1,)), ((), ())),
                         preferred_element_type=jnp.float32) + bih_ref[...]
    gh = lax.dot_general(h, whh_ref[...], (((1,), (1,)), ((), ())),
                         preferred_element_type=jnp.float32) + bhh_ref[...]
    i_r, i_z, i_n = gi[:, :C], gi[:, C:2 * C], gi[:, 2 * C:]
    h_r, h_z, h_n = gh[:, :C], gh[:, C:2 * C], gh[:, 2 * C:]
    r = jax.nn.sigmoid(i_r + h_r)
    z = jax.nn.sigmoid(i_z + h_z)
    n = jnp.tanh(i_n + r * h_n)
    out_ref[...] = (1.0 - z) * n + z * h


def _gru_tc(qs, h, W_ih, W_hh, bih, bhh):
    return pl.pallas_call(
        _gru_body,
        out_shape=jax.ShapeDtypeStruct((S, C), jnp.float32),
    )(qs, h, W_ih, W_hh, bih, bhh)


def _sc_pool_kernel(x_hbm, off_hbm, q_hbm, out_hbm, off_v, q_v, xbuf0, xbuf1,
                    rbuf, sem0, sem1):
    wid = lax.axis_index("s") * 2 + lax.axis_index("c")

    pltpu.sync_copy(off_hbm.at[pl.ds(wid * SPW, 32)], off_v)
    pltpu.sync_copy(q_hbm.at[pl.ds(wid * (SPW * C), SPW * C)], q_v)

    zero16 = jnp.zeros((16,), jnp.float32)
    for j in range(SPW * 8):
        rbuf[pl.ds(j * 16, 16)] = zero16

    v0 = off_v[pl.ds(0, 16)]
    v1 = off_v[pl.ds(16, 16)]
    a = v0[0]
    bend = v1[0]

    def off_at(s):
        return off_v[pl.ds(jnp.minimum(s, 16), 16)][0]

    def q_load(sl):
        sc = jnp.minimum(sl, 15)
        return tuple(q_v[pl.ds(sc * C + 16 * k, 16)] for k in range(8))

    def start_copy(ci, buf, sem):
        cs = a + ci * CH
        csc = jnp.minimum(cs, N_ROWS - CH)
        pltpu.make_async_copy(x_hbm.at[pl.ds(csc * C, CH * C)], buf, sem).start()

    def wait_copy(buf, sem):
        pltpu.make_async_copy(x_hbm.at[pl.ds(0, CH * C)], buf, sem).wait()

    def proc_chunk(ci, xref, st):
        cs = a + ci * CH
        csc = jnp.minimum(cs, N_ROWS - CH)
        ce = jnp.minimum(cs + CH, bend)

        def wcond(s):
            return s[0] < ce

        def wbody(s):
            i, sl, seg_end, m, d, r, q = s
            run_end = jnp.minimum(ce, seg_end)

            def load_row(n):
                base = (n - csc) * C
                return [xref[pl.ds(base + 16 * k, 16)] for k in range(8)]

            def dot(xv):
                p = [xv[k] * q[k] for k in range(8)]
                s0 = p[0] + p[1]
                s1 = p[2] + p[3]
                s2 = p[4] + p[5]
                s3 = p[6] + p[7]
                return jnp.sum((s0 + s1) + (s2 + s3))

            def nbody(n, acc):
                m, d, r = acc
                xv = load_row(n)
                e = dot(xv)
                m_new = jnp.maximum(m, e)
                p = jnp.exp(jnp.broadcast_to(e - m_new, (16,)))
                cc = jnp.exp(jnp.broadcast_to(m - m_new, (16,)))
                d2 = d * cc + p
                r2 = tuple(r[k] * cc + p * xv[k] for k in range(8))
                return (m_new, d2, r2)

            m, d, r = lax.fori_loop(i, run_end, nbody, (m, d, r))

            def adv(args):
                sl, seg_end, m, d, r, q = args
                inv = 1.0 / (d + 1e-16)
                base = jnp.minimum(sl, 15) * C
                for k in range(8):
                    rbuf[pl.ds(base + 16 * k, 16)] = r[k] * inv
                sl2 = sl + 1
                return (sl2, off_at(sl2 + 1), jnp.float32(NEG), zero16,
                        tuple(zero16 for _ in range(8)), q_load(sl2))

            def noadv(args):
                return args

            sl, seg_end, m, d, r, q = lax.cond(
                jnp.logical_and(seg_end <= ce, sl < 16), adv, noadv,
                (sl, seg_end, m, d, r, q))
            return (run_end, sl, seg_end, m, d, r, q)

        return lax.while_loop(wcond, wbody, st)

    nch = lax.div(bend - a + (CH - 1), CH)

    @pl.when(nch > 0)
    def _p0():
        start_copy(0, xbuf0, sem0)

    @pl.when(nch > 1)
    def _p1():
        start_copy(1, xbuf1, sem1)

    st0 = (a, jnp.int32(0), off_at(1), jnp.float32(NEG), zero16,
           tuple(zero16 for _ in range(8)), q_load(jnp.int32(0)))

    def pair_body(kk, st):
        ci0 = 2 * kk
        wait_copy(xbuf0, sem0)
        st = proc_chunk(ci0, xbuf0, st)

        @pl.when(ci0 + 2 < nch)
        def _n0():
            start_copy(ci0 + 2, xbuf0, sem0)

        wait_copy(xbuf1, sem1)
        st = proc_chunk(ci0 + 1, xbuf1, st)

        @pl.when(ci0 + 3 < nch)
        def _n1():
            start_copy(ci0 + 3, xbuf1, sem1)

        return st

    st = lax.fori_loop(0, lax.div(nch, 2), pair_body, st0)

    def tail(s):
        wait_copy(xbuf0, sem0)
        return proc_chunk(nch - 1, xbuf0, s)

    st = lax.cond(lax.rem(nch, 2) == 1, tail, lambda s: s, st)

    pltpu.sync_copy(rbuf, out_hbm.at[pl.ds(wid * (SPW * C), SPW * C)])


def _sc_pool(x1d, off, q):
    mesh = plsc.VectorSubcoreMesh(core_axis_name="c", subcore_axis_name="s")
    fn = functools.partial(
        pl.kernel,
        mesh=mesh,
        compiler_params=pltpu.CompilerParams(needs_layout_passes=False),
        out_type=jax.ShapeDtypeStruct((S * C,), jnp.float32),
        scratch_types=[
            pltpu.VMEM((32,), jnp.int32),
            pltpu.VMEM((SPW * C,), jnp.float32),
            pltpu.VMEM((CH * C,), jnp.float32),
            pltpu.VMEM((CH * C,), jnp.float32),
            pltpu.VMEM((SPW * C,), jnp.float32),
            pltpu.SemaphoreType.DMA,
            pltpu.SemaphoreType.DMA,
        ],
    )(_sc_pool_kernel)
    return fn(x1d, off, q.reshape(S * C)).reshape(S, C)


@jax.jit
def kernel(x, batch, W_ih, W_hh, b_ih, b_hh):
    batch_p = jnp.pad(batch.astype(jnp.int32), (0, NPAD - N_ROWS),
                      constant_values=S)
    off = _sc_offsets(batch_p)

    x1d = x.reshape(-1)
    bih = b_ih.reshape(1, 3 * C)
    bhh = b_hh.reshape(1, 3 * C)

    h = jnp.zeros((S, C), jnp.float32)
    qs = jnp.zeros((S, 2 * C), jnp.float32)
    for _ in range(STEPS):
        h = _gru_tc(qs, h, W_ih, W_hh, bih, bhh)
        r = _sc_pool(x1d, off, h)
        qs = jnp.concatenate([h, r], axis=-1)
    return qs


# speedup vs baseline: 1.0457x; 1.0457x over previous
import functools
import jax
import jax.numpy as jnp
from jax import lax
from jax.experimental import pallas as pl
from jax.experimental.pallas import tpu as pltpu
from jax.experimental.pallas import tpu_sc as plsc

C = 128
S = 512
STEPS = 3
T = 512
NW = 32
SPW = S // NW
CH = 256
N_ROWS = 100000
NEG = -1e30


SLICE = 3136
NPAD = 32 * SLICE
NVEC = SLICE // 16


def _sc_off_scan(b_hbm, bnd_hbm, ibuf, bnd):
    wid = lax.axis_index("s") * 2 + lax.axis_index("c")
    lane = lax.broadcasted_iota(jnp.int32, (16,), 0)
    nfull = jnp.full((16,), N_ROWS, jnp.int32)
    for j in range(34):
        bnd[pl.ds(16 * j, 16)] = nfull

    base0 = SLICE * wid - 16

    @pl.when(wid > 0)
    def _ld():
        pltpu.sync_copy(b_hbm.at[pl.ds(base0, 16 + SLICE)], ibuf)

    @pl.when(wid == 0)
    def _ld0():
        ibuf[pl.ds(0, 16)] = jnp.full((16,), -1, jnp.int32)
        pltpu.sync_copy(b_hbm.at[pl.ds(0, SLICE)], ibuf.at[pl.ds(16, SLICE)])

    def vec(j, _):
        v = ibuf[pl.ds(16 + j * 16, 16)]
        sv = ibuf[pl.ds(15 + j * 16, 16)]
        pos = jnp.broadcast_to(base0 + 16 + j * 16, (16,)).astype(jnp.int32) + lane
        plsc.store_scatter(bnd, [v], pos, mask=v != sv)
        return 0

    lax.fori_loop(0, NVEC, vec, 0)
    pltpu.sync_copy(bnd, bnd_hbm.at[wid])


def _sc_off_combine(bnd_hbm, off_hbm, cmb, offv):
    wid = lax.axis_index("s") * 2 + lax.axis_index("c")

    @pl.when(wid == 0)
    def _combine():
        pltpu.sync_copy(bnd_hbm, cmb)
        carry = jnp.int32(N_ROWS)
        for j in reversed(range(34)):
            v = cmb[0, pl.ds(16 * j, 16)]
            for row in range(1, 32):
                v = jnp.minimum(v, cmb[row, pl.ds(16 * j, 16)])
            pm = -plsc.cummax(-lax.rev(v, (0,)))
            pm2 = jnp.minimum(pm, jnp.broadcast_to(carry, (16,)))
            carry = pm2[15]
            offv[pl.ds(16 * j, 16)] = lax.rev(pm2, (0,))
        pltpu.sync_copy(offv, off_hbm)


def _sc_offsets(batch_p):
    mesh = plsc.VectorSubcoreMesh(core_axis_name="c", subcore_axis_name="s")
    scan = functools.partial(
        pl.kernel,
        mesh=mesh,
        compiler_params=pltpu.CompilerParams(needs_layout_passes=False),
        out_type=jax.ShapeDtypeStruct((32, 544), jnp.int32),
        scratch_types=[
            pltpu.VMEM((16 + SLICE,), jnp.int32),
            pltpu.VMEM((544,), jnp.int32),
        ],
    )(_sc_off_scan)
    combine = functools.partial(
        pl.kernel,
        mesh=mesh,
        compiler_params=pltpu.CompilerParams(needs_layout_passes=False),
        out_type=jax.ShapeDtypeStruct((544,), jnp.int32),
        scratch_types=[
            pltpu.VMEM((32, 544), jnp.int32),
            pltpu.VMEM((544,), jnp.int32),
        ],
    )(_sc_off_combine)
    return combine(scan(batch_p))


def _gru_body(qs_ref, h_ref, wih_ref, whh_ref, bih_ref, bhh_ref, out_ref):
    qs, h = qs_ref[...], h_ref[...]
    gi = lax.dot_general(qs, wih_ref[...], (((1,), (1,)), ((), ())),
                         preferred_element_type=jnp.float32) + bih_ref[...]
    gh = lax.dot_general(h, whh_ref[...], (((1,), (1,)), ((), ())),
                         preferred_element_type=jnp.float32) + bhh_ref[...]
    i_r, i_z, i_n = gi[:, :C], gi[:, C:2 * C], gi[:, 2 * C:]
    h_r, h_z, h_n = gh[:, :C], gh[:, C:2 * C], gh[:, 2 * C:]
    r = jax.nn.sigmoid(i_r + h_r)
    z = jax.nn.sigmoid(i_z + h_z)
    n = jnp.tanh(i_n + r * h_n)
    out_ref[...] = (1.0 - z) * n + z * h


def _gru_tc(qs, h, W_ih, W_hh, bih, bhh):
    return pl.pallas_call(
        _gru_body,
        out_shape=jax.ShapeDtypeStruct((S, C), jnp.float32),
    )(qs, h, W_ih, W_hh, bih, bhh)


def _sc_pool_kernel(x_hbm, off_hbm, q_hbm, out_hbm, off_v, q_v, xbuf0, xbuf1,
                    rbuf, sem0, sem1):
    wid = lax.axis_index("s") * 2 + lax.axis_index("c")

    pltpu.sync_copy(off_hbm.at[pl.ds(wid * SPW, 32)], off_v)
    pltpu.sync_copy(q_hbm.at[pl.ds(wid * (SPW * C), SPW * C)], q_v)

    zero16 = jnp.zeros((16,), jnp.float32)
    for j in range(SPW * 8):
        rbuf[pl.ds(j * 16, 16)] = zero16

    v0 = off_v[pl.ds(0, 16)]
    v1 = off_v[pl.ds(16, 16)]
    a = v0[0]
    bend = v1[0]

    def off_at(s):
        return off_v[pl.ds(jnp.minimum(s, 16), 16)][0]

    def q_load(sl):
        sc = jnp.minimum(sl, 15)
        return tuple(q_v[pl.ds(sc * C + 16 * k, 16)] for k in range(8))

    def start_copy(ci, buf, sem):
        cs = a + ci * CH
        csc = jnp.minimum(cs, N_ROWS - CH)
        pltpu.make_async_copy(x_hbm.at[pl.ds(csc * C, CH * C)], buf, sem).start()

    def wait_copy(buf, sem):
        pltpu.make_async_copy(x_hbm.at[pl.ds(0, CH * C)], buf, sem).wait()

    def proc_chunk(ci, xref, st):
        cs = a + ci * CH
        csc = jnp.minimum(cs, N_ROWS - CH)
        ce = jnp.minimum(cs + CH, bend)

        def wcond(s):
            return s[0] < ce

        def wbody(s):
            i, sl, seg_end, m, d, r, q = s
            run_end = jnp.minimum(ce, seg_end)

            def load_row(n):
                base = (n - csc) * C
                return [xref[pl.ds(base + 16 * k, 16)] for k in range(8)]

            def dot(xv):
                dv = xv[0] * q[0]
                for k in range(1, 8):
                    dv = dv + xv[k] * q[k]
                return jnp.sum(dv)

            def nbody(n, acc):
                m, d, r = acc
                e = dot(load_row(n))
                m_new = jnp.maximum(m, e)
                p = jnp.exp(jnp.broadcast_to(e - m_new, (16,)))
                cc = jnp.exp(jnp.broadcast_to(m - m_new, (16,)))
                d2 = d * cc + p
                xv = load_row(n)
                r2 = tuple(r[k] * cc + p * xv[k] for k in range(8))
                return (m_new, d2, r2)

            m, d, r = lax.fori_loop(i, run_end, nbody, (m, d, r))

            def adv(args):
                sl, seg_end, m, d, r, q = args
                inv = 1.0 / (d + 1e-16)
                base = jnp.minimum(sl, 15) * C
                for k in range(8):
                    rbuf[pl.ds(base + 16 * k, 16)] = r[k] * inv
                sl2 = sl + 1
                return (sl2, off_at(sl2 + 1), jnp.float32(NEG), zero16,
                        tuple(zero16 for _ in range(8)), q_load(sl2))

            def noadv(args):
                return args

            sl, seg_end, m, d, r, q = lax.cond(
                jnp.logical_and(seg_end <= ce, sl < 16), adv, noadv,
                (sl, seg_end, m, d, r, q))
            return (run_end, sl, seg_end, m, d, r, q)

        return lax.while_loop(wcond, wbody, st)

    nch = lax.div(bend - a + (CH - 1), CH)

    @pl.when(nch > 0)
    def _p0():
        start_copy(0, xbuf0, sem0)

    @pl.when(nch > 1)
    def _p1():
        start_copy(1, xbuf1, sem1)

    st0 = (a, jnp.int32(0), off_at(1), jnp.float32(NEG), zero16,
           tuple(zero16 for _ in range(8)), q_load(jnp.int32(0)))

    def pair_body(kk, st):
        ci0 = 2 * kk
        wait_copy(xbuf0, sem0)
        st = proc_chunk(ci0, xbuf0, st)

        @pl.when(ci0 + 2 < nch)
        def _n0():
            start_copy(ci0 + 2, xbuf0, sem0)

        wait_copy(xbuf1, sem1)
        st = proc_chunk(ci0 + 1, xbuf1, st)

        @pl.when(ci0 + 3 < nch)
        def _n1():
            start_copy(ci0 + 3, xbuf1, sem1)

        return st

    st = lax.fori_loop(0, lax.div(nch, 2), pair_body, st0)

    def tail(s):
        wait_copy(xbuf0, sem0)
        return proc_chunk(nch - 1, xbuf0, s)

    st = lax.cond(lax.rem(nch, 2) == 1, tail, lambda s: s, st)

    pltpu.sync_copy(rbuf, out_hbm.at[pl.ds(wid * (SPW * C), SPW * C)])


def _sc_pool(x1d, off, q):
    mesh = plsc.VectorSubcoreMesh(core_axis_name="c", subcore_axis_name="s")
    fn = functools.partial(
        pl.kernel,
        mesh=mesh,
        compiler_params=pltpu.CompilerParams(needs_layout_passes=False),
        out_type=jax.ShapeDtypeStruct((S * C,), jnp.float32),
        scratch_types=[
            pltpu.VMEM((32,), jnp.int32),
            pltpu.VMEM((SPW * C,), jnp.float32),
            pltpu.VMEM((CH * C,), jnp.float32),
            pltpu.VMEM((CH * C,), jnp.float32),
            pltpu.VMEM((SPW * C,), jnp.float32),
            pltpu.SemaphoreType.DMA,
            pltpu.SemaphoreType.DMA,
        ],
    )(_sc_pool_kernel)
    return fn(x1d, off, q.reshape(S * C)).reshape(S, C)


@jax.jit
def kernel(x, batch, W_ih, W_hh, b_ih, b_hh):
    batch_p = jnp.pad(batch.astype(jnp.int32), (0, NPAD - N_ROWS),
                      constant_values=S)
    off = _sc_offsets(batch_p)

    x1d = x.reshape(-1)
    bih = b_ih.reshape(1, 3 * C)
    bhh = b_hh.reshape(1, 3 * C)

    h = jnp.zeros((S, C), jnp.float32)
    qs = jnp.zeros((S, 2 * C), jnp.float32)
    for _ in range(STEPS):
        h = _gru_tc(qs, h, W_ih, W_hh, bih, bhh)
        r = _sc_pool(x1d, off, h)
        qs = jnp.concatenate([h, r], axis=-1)
    return qs


# speedup vs baseline: 1.0651x; 1.0185x over previous
import functools
import jax
import jax.numpy as jnp
from jax import lax
from jax.experimental import pallas as pl
from jax.experimental.pallas import tpu as pltpu
from jax.experimental.pallas import tpu_sc as plsc

C = 128
S = 512
STEPS = 3
T = 512
NW = 32
SPW = S // NW
CH = 256
N_ROWS = 100000
NEG = -1e30


SLICE = 3136
NPAD = 32 * SLICE
NVEC = SLICE // 16


def _sc_off_scan(b_hbm, bnd_hbm, ibuf, bnd):
    wid = lax.axis_index("s") * 2 + lax.axis_index("c")
    lane = lax.broadcasted_iota(jnp.int32, (16,), 0)
    nfull = jnp.full((16,), N_ROWS, jnp.int32)
    for j in range(34):
        bnd[pl.ds(16 * j, 16)] = nfull

    base0 = SLICE * wid - 16

    @pl.when(wid > 0)
    def _ld():
        pltpu.sync_copy(b_hbm.at[pl.ds(base0, 16 + SLICE)], ibuf)

    @pl.when(wid == 0)
    def _ld0():
        ibuf[pl.ds(0, 16)] = jnp.full((16,), -1, jnp.int32)
        pltpu.sync_copy(b_hbm.at[pl.ds(0, SLICE)], ibuf.at[pl.ds(16, SLICE)])

    def vec(j, _):
        v = ibuf[pl.ds(16 + j * 16, 16)]
        sv = ibuf[pl.ds(15 + j * 16, 16)]
        pos = jnp.broadcast_to(base0 + 16 + j * 16, (16,)).astype(jnp.int32) + lane
        plsc.store_scatter(bnd, [v], pos, mask=v != sv)
        return 0

    lax.fori_loop(0, NVEC, vec, 0)
    pltpu.sync_copy(bnd, bnd_hbm.at[wid])


def _sc_off_combine(bnd_hbm, off_hbm, cmb, offv):
    wid = lax.axis_index("s") * 2 + lax.axis_index("c")

    @pl.when(wid == 0)
    def _combine():
        pltpu.sync_copy(bnd_hbm, cmb)
        carry = jnp.int32(N_ROWS)
        for j in reversed(range(34)):
            v = cmb[0, pl.ds(16 * j, 16)]
            for row in range(1, 32):
                v = jnp.minimum(v, cmb[row, pl.ds(16 * j, 16)])
            pm = -plsc.cummax(-lax.rev(v, (0,)))
            pm2 = jnp.minimum(pm, jnp.broadcast_to(carry, (16,)))
            carry = pm2[15]
            offv[pl.ds(16 * j, 16)] = lax.rev(pm2, (0,))
        pltpu.sync_copy(offv, off_hbm)


def _sc_offsets(batch_p):
    mesh = plsc.VectorSubcoreMesh(core_axis_name="c", subcore_axis_name="s")
    scan = functools.partial(
        pl.kernel,
        mesh=mesh,
        compiler_params=pltpu.CompilerParams(needs_layout_passes=False),
        out_type=jax.ShapeDtypeStruct((32, 544), jnp.int32),
        scratch_types=[
            pltpu.VMEM((16 + SLICE,), jnp.int32),
            pltpu.VMEM((544,), jnp.int32),
        ],
    )(_sc_off_scan)
    combine = functools.partial(
        pl.kernel,
        mesh=mesh,
        compiler_params=pltpu.CompilerParams(needs_layout_passes=False),
        out_type=jax.ShapeDtypeStruct((544,), jnp.int32),
        scratch_types=[
            pltpu.VMEM((32, 544), jnp.int32),
            pltpu.VMEM((544,), jnp.int32),
        ],
    )(_sc_off_combine)
    return combine(scan(batch_p))


def _gru_body(q_ref, r_ref, h_ref, w1_ref, w2_ref, whh_ref, bih_ref,
              bhh_ref, out_ref):
    h = h_ref[...]
    gi = (lax.dot_general(q_ref[...], w1_ref[...], (((1,), (1,)), ((), ())),
                          preferred_element_type=jnp.float32)
          + lax.dot_general(r_ref[...], w2_ref[...], (((1,), (1,)), ((), ())),
                            preferred_element_type=jnp.float32)
          + bih_ref[...])
    gh = lax.dot_general(h, whh_ref[...], (((1,), (1,)), ((), ())),
                         preferred_element_type=jnp.float32) + bhh_ref[...]
    i_r, i_z, i_n = gi[:, :C], gi[:, C:2 * C], gi[:, 2 * C:]
    h_r, h_z, h_n = gh[:, :C], gh[:, C:2 * C], gh[:, 2 * C:]
    r = jax.nn.sigmoid(i_r + h_r)
    z = jax.nn.sigmoid(i_z + h_z)
    n = jnp.tanh(i_n + r * h_n)
    out_ref[...] = (1.0 - z) * n + z * h


def _gru_tc(q, r, h, W1, W2, W_hh, bih, bhh):
    return pl.pallas_call(
        _gru_body,
        out_shape=jax.ShapeDtypeStruct((S, C), jnp.float32),
    )(q, r, h, W1, W2, W_hh, bih, bhh)


def _sc_pool_kernel(x_hbm, off_hbm, q_hbm, out_hbm, off_v, q_v, xbuf0, xbuf1,
                    rbuf, sem0, sem1):
    wid = lax.axis_index("s") * 2 + lax.axis_index("c")

    pltpu.sync_copy(off_hbm.at[pl.ds(wid * SPW, 32)], off_v)
    pltpu.sync_copy(q_hbm.at[pl.ds(wid * (SPW * C), SPW * C)], q_v)

    zero16 = jnp.zeros((16,), jnp.float32)
    for j in range(SPW * 8):
        rbuf[pl.ds(j * 16, 16)] = zero16

    v0 = off_v[pl.ds(0, 16)]
    v1 = off_v[pl.ds(16, 16)]
    a = v0[0]
    bend = v1[0]

    def off_at(s):
        return off_v[pl.ds(jnp.minimum(s, 16), 16)][0]

    def q_load(sl):
        sc = jnp.minimum(sl, 15)
        return tuple(q_v[pl.ds(sc * C + 16 * k, 16)] for k in range(8))

    def start_copy(ci, buf, sem):
        cs = a + ci * CH
        csc = jnp.minimum(cs, N_ROWS - CH)
        pltpu.make_async_copy(x_hbm.at[pl.ds(csc * C, CH * C)], buf, sem).start()

    def wait_copy(buf, sem):
        pltpu.make_async_copy(x_hbm.at[pl.ds(0, CH * C)], buf, sem).wait()

    def proc_chunk(ci, xref, st):
        cs = a + ci * CH
        csc = jnp.minimum(cs, N_ROWS - CH)
        ce = jnp.minimum(cs + CH, bend)

        def wcond(s):
            return s[0] < ce

        def wbody(s):
            i, sl, seg_end, m, d, r, q = s
            run_end = jnp.minimum(ce, seg_end)

            def load_row(n):
                base = (n - csc) * C
                return [xref[pl.ds(base + 16 * k, 16)] for k in range(8)]

            def dot(xv):
                dv = xv[0] * q[0]
                for k in range(1, 8):
                    dv = dv + xv[k] * q[k]
                return jnp.sum(dv)

            def nbody(n, acc):
                m, d, r = acc
                e = dot(load_row(n))
                m_new = jnp.maximum(m, e)
                p = jnp.exp(jnp.broadcast_to(e - m_new, (16,)))
                cc = jnp.exp(jnp.broadcast_to(m - m_new, (16,)))
                d2 = d * cc + p
                xv = load_row(n)
                r2 = tuple(r[k] * cc + p * xv[k] for k in range(8))
                return (m_new, d2, r2)

            m, d, r = lax.fori_loop(i, run_end, nbody, (m, d, r))

            def adv(args):
                sl, seg_end, m, d, r, q = args
                inv = 1.0 / (d + 1e-16)
                base = jnp.minimum(sl, 15) * C
                for k in range(8):
                    rbuf[pl.ds(base + 16 * k, 16)] = r[k] * inv
                sl2 = sl + 1
                return (sl2, off_at(sl2 + 1), jnp.float32(NEG), zero16,
                        tuple(zero16 for _ in range(8)), q_load(sl2))

            def noadv(args):
                return args

            sl, seg_end, m, d, r, q = lax.cond(
                jnp.logical_and(seg_end <= ce, sl < 16), adv, noadv,
                (sl, seg_end, m, d, r, q))
            return (run_end, sl, seg_end, m, d, r, q)

        return lax.while_loop(wcond, wbody, st)

    nch = lax.div(bend - a + (CH - 1), CH)

    @pl.when(nch > 0)
    def _p0():
        start_copy(0, xbuf0, sem0)

    @pl.when(nch > 1)
    def _p1():
        start_copy(1, xbuf1, sem1)

    st0 = (a, jnp.int32(0), off_at(1), jnp.float32(NEG), zero16,
           tuple(zero16 for _ in range(8)), q_load(jnp.int32(0)))

    def pair_body(kk, st):
        ci0 = 2 * kk
        wait_copy(xbuf0, sem0)
        st = proc_chunk(ci0, xbuf0, st)

        @pl.when(ci0 + 2 < nch)
        def _n0():
            start_copy(ci0 + 2, xbuf0, sem0)

        wait_copy(xbuf1, sem1)
        st = proc_chunk(ci0 + 1, xbuf1, st)

        @pl.when(ci0 + 3 < nch)
        def _n1():
            start_copy(ci0 + 3, xbuf1, sem1)

        return st

    st = lax.fori_loop(0, lax.div(nch, 2), pair_body, st0)

    def tail(s):
        wait_copy(xbuf0, sem0)
        return proc_chunk(nch - 1, xbuf0, s)

    st = lax.cond(lax.rem(nch, 2) == 1, tail, lambda s: s, st)

    pltpu.sync_copy(rbuf, out_hbm.at[pl.ds(wid * (SPW * C), SPW * C)])


def _sc_pool(x1d, off, q):
    mesh = plsc.VectorSubcoreMesh(core_axis_name="c", subcore_axis_name="s")
    fn = functools.partial(
        pl.kernel,
        mesh=mesh,
        compiler_params=pltpu.CompilerParams(needs_layout_passes=False),
        out_type=jax.ShapeDtypeStruct((S * C,), jnp.float32),
        scratch_types=[
            pltpu.VMEM((32,), jnp.int32),
            pltpu.VMEM((SPW * C,), jnp.float32),
            pltpu.VMEM((CH * C,), jnp.float32),
            pltpu.VMEM((CH * C,), jnp.float32),
            pltpu.VMEM((SPW * C,), jnp.float32),
            pltpu.SemaphoreType.DMA,
            pltpu.SemaphoreType.DMA,
        ],
    )(_sc_pool_kernel)
    return fn(x1d, off, q.reshape(S * C)).reshape(S, C)


@jax.jit
def kernel(x, batch, W_ih, W_hh, b_ih, b_hh):
    batch_p = jnp.pad(batch.astype(jnp.int32), (0, NPAD - N_ROWS),
                      constant_values=S)
    off = _sc_offsets(batch_p)

    x1d = x.reshape(-1)
    bih = b_ih.reshape(1, 3 * C)
    bhh = b_hh.reshape(1, 3 * C)
    W1 = W_ih[:, :C]
    W2 = W_ih[:, C:]

    h = jnp.zeros((S, C), jnp.float32)
    r = jnp.zeros((S, C), jnp.float32)
    for _ in range(STEPS):
        h = _gru_tc(h, r, h, W1, W2, W_hh, bih, bhh)
        r = _sc_pool(x1d, off, h)
    return jnp.concatenate([h, r], axis=-1)
